# Initial kernel scaffold; baseline (speedup 1.0000x reference)
import jax
import jax.numpy as jnp
from jax.experimental import pallas as pl


def kernel(x, edge_index, batch, W0, b0, W1, b1, W2, b2):
    raise NotImplementedError("write your pallas kernel here")


# speedup vs baseline: 5.8242x; 5.8242x over previous
import jax
import jax.numpy as jnp
from jax.experimental import pallas as pl
from jax.experimental.pallas import tpu as pltpu

N = 10000
D_IN = 128
D_H = 256
G = 512
BLK = 1000
GRID = N // BLK


def _fused_body(x_ref, b_ref, W0_ref, b0_ref, W1_ref, b1_ref, w2_ref, b2_ref,
                out_ref, acc_ref):
    i = pl.program_id(0)

    h = jnp.dot(x_ref[...], W0_ref[...], preferred_element_type=jnp.float32)
    h = h + b0_ref[...]
    h = jnp.where(h > 0, h, jnp.exp(jnp.minimum(h, 0.0)) - 1.0)

    gids = jax.lax.broadcasted_iota(jnp.int32, (BLK, G), 1)
    onehot = (b_ref[...] == gids).astype(jnp.float32)
    part = jax.lax.dot_general(
        onehot, h, ((( 0,), (0,)), ((), ())),
        preferred_element_type=jnp.float32)

    @pl.when(i == 0)
    def _init():
        acc_ref[...] = part

    @pl.when(i > 0)
    def _acc():
        acc_ref[...] += part

    @pl.when(i == GRID - 1)
    def _head():
        pooled = acc_ref[...]
        h2 = jnp.dot(pooled, W1_ref[...], preferred_element_type=jnp.float32)
        h2 = jnp.maximum(h2 + b1_ref[...], 0.0)
        logit = jnp.sum(h2 * w2_ref[...], axis=1, keepdims=True) + b2_ref[...]
        out_ref[...] = 1.0 / (1.0 + jnp.exp(-logit))


def kernel(x, edge_index, batch, W0, b0, W1, b1, W2, b2):
    del edge_index
    batch2d = batch.reshape(N, 1)
    out = pl.pallas_call(
        _fused_body,
        grid=(GRID,),
        in_specs=[
            pl.BlockSpec((BLK, D_IN), lambda i: (i, 0)),
            pl.BlockSpec((BLK, 1), lambda i: (i, 0)),
            pl.BlockSpec((D_IN, D_H), lambda i: (0, 0)),
            pl.BlockSpec((1, D_H), lambda i: (0, 0)),
            pl.BlockSpec((D_H, D_H), lambda i: (0, 0)),
            pl.BlockSpec((1, D_H), lambda i: (0, 0)),
            pl.BlockSpec((1, D_H), lambda i: (0, 0)),
            pl.BlockSpec((1, 1), lambda i: (0, 0)),
        ],
        out_specs=pl.BlockSpec((G, 1), lambda i: (0, 0)),
        out_shape=jax.ShapeDtypeStruct((G, 1), jnp.float32),
        scratch_shapes=[pltpu.VMEM((G, D_H), jnp.float32)],
    )(x, batch2d, W0, b0.reshape(1, D_H), W1, b1.reshape(1, D_H),
      W2.reshape(1, D_H), b2.reshape(1, 1))
    return out.reshape(G)
